# baseline (speedup 1.0000x reference)
import functools

import jax
import jax.numpy as jnp
from jax import lax
from jax.experimental import pallas as pl
from jax.experimental.pallas import tpu as pltpu
from jax.experimental.pallas import tpu_sc as plsc

N = 10000
E = 320000
D_N = 128
D_E = 128
D_F = 16
H = 256
D_OUT = 128

NC = 2
NS = 16
CHUNK = 128
ROWS_PER_TILE = 624
TAIL_ROWS = N - NS * ROWS_PER_TILE

NCHUNKS = E // CHUNK
ZROWS = 640

WB_CHUNKS = 1875
N_WB_TILES = 26
SUF_ROWS = (NCHUNKS - WB_CHUNKS) * CHUNK


def _segment_sum_sc(efeat, edge_index, zeros):
    mesh = plsc.VectorSubcoreMesh(
        core_axis_name="c", subcore_axis_name="s", num_cores=NC, num_subcores=NS
    )

    @functools.partial(
        pl.kernel,
        out_type=(
            jax.ShapeDtypeStruct((NC, N, D_E), jnp.float32),
            jax.ShapeDtypeStruct((E, D_E), jnp.float32),
        ),
        mesh=mesh,
        scratch_types=[
            pltpu.VMEM_SHARED((N, D_E), jnp.float32),
            pltpu.VMEM((CHUNK, D_E), jnp.float32),
            pltpu.VMEM((CHUNK, D_E), jnp.float32),
            pltpu.VMEM((2, CHUNK), jnp.int32),
            pltpu.VMEM((2, CHUNK), jnp.int32),
            pltpu.SemaphoreType.DMA,
            pltpu.SemaphoreType.DMA,
            pltpu.SemaphoreType.DMA,
            pltpu.SemaphoreType.DMA,
            pltpu.SemaphoreType.DMA,
            pltpu.SemaphoreType.DMA,
            pltpu.SemaphoreType.DMA,
            pltpu.SemaphoreType.DMA,
        ],
    )
    def k(efeat_hbm, ei_hbm, z_hbm, out_hbm, eout_hbm, acc,
          rows0, rows1, idx0, idx1, sr0, sr1, si0, si1, sw0, sw1, ss0, ss1):
        cid = lax.axis_index("c")
        sid = lax.axis_index("s")
        w = cid * NS + sid
        r0 = sid * ROWS_PER_TILE
        pltpu.sync_copy(
            z_hbm.at[pl.ds(0, ROWS_PER_TILE)], acc.at[pl.ds(r0, ROWS_PER_TILE)]
        )

        @pl.when(sid == NS - 1)
        def _():
            pltpu.sync_copy(
                z_hbm.at[pl.ds(0, TAIL_ROWS)],
                acc.at[pl.ds(NS * ROWS_PER_TILE, TAIL_ROWS)],
            )

        wb_on = w < N_WB_TILES
        nc_w = jnp.where(
            w < 3, 73, jnp.where(w < 26, 72, jnp.where(w < 31, 104, 105))
        )
        c0_w = jnp.where(
            w < 26, 72 * w + jnp.minimum(w, 3), WB_CHUNKS + 104 * (w - 26)
        )
        eb = c0_w * CHUNK

        def start(i, rows, idx, sr, si):
            b = eb + i * CHUNK
            pltpu.async_copy(efeat_hbm.at[pl.ds(b, CHUNK)], rows, sr)
            pltpu.async_copy(ei_hbm.at[pl.ds(0, 2), pl.ds(b, CHUNK)], idx, si)

        def wait(rows, idx, sr, si):
            pltpu.make_async_copy(efeat_hbm.at[pl.ds(0, CHUNK)], rows, sr).wait()
            pltpu.make_async_copy(
                ei_hbm.at[pl.ds(0, 2), pl.ds(0, CHUNK)], idx, si
            ).wait()

        def sc_start(rows, idx, ss):
            pltpu.async_copy(rows, acc.at[idx.at[1]], ss, add=True)

        def sc_wait(rows, idx, ss):
            pltpu.make_async_copy(rows, acc.at[idx.at[1]], ss).wait()

        def wb_start(i, rows, sw):
            @pl.when(wb_on)
            def _():
                pltpu.async_copy(
                    rows, eout_hbm.at[pl.ds(eb + i * CHUNK, CHUNK)], sw
                )

        def wb_wait(rows, sw):
            @pl.when(wb_on)
            def _():
                pltpu.make_async_copy(
                    rows, eout_hbm.at[pl.ds(0, CHUNK)], sw
                ).wait()

        start(0, rows0, idx0, sr0, si0)
        plsc.subcore_barrier()

        @pl.loop(0, nc_w // 2)
        def _(j):
            i0 = 2 * j

            @pl.when(j > 0)
            def _():
                sc_wait(rows1, idx1, ss1)
                wb_wait(rows1, sw1)

            start(i0 + 1, rows1, idx1, sr1, si1)
            wait(rows0, idx0, sr0, si0)
            sc_start(rows0, idx0, ss0)
            wb_start(i0, rows0, sw0)
            wait(rows1, idx1, sr1, si1)
            sc_start(rows1, idx1, ss1)
            wb_start(i0 + 1, rows1, sw1)
            sc_wait(rows0, idx0, ss0)
            wb_wait(rows0, sw0)

            @pl.when(i0 + 2 < nc_w)
            def _():
                start(i0 + 2, rows0, idx0, sr0, si0)

        sc_wait(rows1, idx1, ss1)
        wb_wait(rows1, sw1)

        @pl.when(nc_w % 2 == 1)
        def _():
            wait(rows0, idx0, sr0, si0)
            sc_start(rows0, idx0, ss0)
            wb_start(nc_w - 1, rows0, sw0)
            sc_wait(rows0, idx0, ss0)
            wb_wait(rows0, sw0)

        plsc.subcore_barrier()
        pltpu.sync_copy(
            acc.at[pl.ds(r0, ROWS_PER_TILE)],
            out_hbm.at[cid, pl.ds(r0, ROWS_PER_TILE)],
        )

        @pl.when(sid == NS - 1)
        def _():
            pltpu.sync_copy(
                acc.at[pl.ds(NS * ROWS_PER_TILE, TAIL_ROWS)],
                out_hbm.at[cid, pl.ds(NS * ROWS_PER_TILE, TAIL_ROWS)],
            )

    return k(efeat, edge_index, zeros)


BR = 400
SUF_BLOCK = SUF_ROWS // (N // BR)
SUF_OFF = (E - SUF_ROWS) // SUF_BLOCK


def _ln(x, g, b, eps=1e-5):
    mu = jnp.mean(x, axis=-1, keepdims=True)
    var = jnp.mean((x - mu) ** 2, axis=-1, keepdims=True)
    return (x - mu) / jnp.sqrt(var + eps) * g + b


def _silu(x):
    return x / (1.0 + jnp.exp(-x))


def _dot3(x, wh, wl):
    xh = x.astype(jnp.bfloat16)
    xl = (x - xh.astype(jnp.float32)).astype(jnp.bfloat16)
    d = lambda a, b: jax.lax.dot_general(
        a, b, (((1,), (0,)), ((), ())), preferred_element_type=jnp.float32
    )
    return d(xh, wh) + d(xh, wl) + d(xl, wh)


def _mlp_body(nf, agg2, fl, ef_suf, ef_sc, w1nh, w1nl, w1eh, w1el, w1fh, w1fl,
              b1, g1, be1, w2h, w2l, b2, g2, be2, w3h, w3l, b3, out, ef_out):
    del ef_sc
    ef_out[...] = ef_suf[...]
    x_n = nf[...]
    agg = agg2[0] + agg2[1]
    h = (
        _dot3(x_n, w1nh[...], w1nl[...])
        + _dot3(agg, w1eh[...], w1el[...])
        + _dot3(fl[...], w1fh[...], w1fl[...])
        + b1[...]
    )
    h = _silu(_ln(h, g1[...], be1[...]))
    h = _dot3(h, w2h[...], w2l[...]) + b2[...]
    h = _silu(_ln(h, g2[...], be2[...]))
    out[...] = _dot3(h, w3h[...], w3l[...]) + b3[...] + x_n


def _mlp_tc(nfeat, agg2, flow, efeat, efeat_sc, w1h, w1l, b1, g1, be1,
            w2h, w2l, b2, g2, be2, w3h, w3l, b3):
    row_block = lambda d: pl.BlockSpec((BR, d), lambda i: (i, 0))
    full = lambda s: pl.BlockSpec(s, lambda i: (0, 0))
    suf_block = pl.BlockSpec((SUF_BLOCK, D_E), lambda i: (i + SUF_OFF, 0))
    w1_slabs = [
        pl.BlockSpec((D_N, H), lambda i: (0, 0)),
        pl.BlockSpec((D_E, H), lambda i: (1, 0)),
        pl.BlockSpec((D_F, H), lambda i: ((D_N + D_E) // D_F, 0)),
    ]
    return pl.pallas_call(
        _mlp_body,
        grid=(N // BR,),
        in_specs=[
            row_block(D_N),
            pl.BlockSpec((NC, BR, D_E), lambda i: (0, i, 0)),
            row_block(D_F),
            suf_block,
            pl.BlockSpec(memory_space=pl.ANY),
            w1_slabs[0], w1_slabs[0],
            w1_slabs[1], w1_slabs[1],
            w1_slabs[2], w1_slabs[2],
            full((1, H)),
            full((1, H)),
            full((1, H)),
            full((H, H)), full((H, H)),
            full((1, H)),
            full((1, H)),
            full((1, H)),
            full((H, D_OUT)), full((H, D_OUT)),
            full((1, D_OUT)),
        ],
        out_specs=[row_block(D_OUT), suf_block],
        out_shape=[
            jax.ShapeDtypeStruct((N, D_OUT), jnp.float32),
            jax.ShapeDtypeStruct((E, D_E), jnp.float32),
        ],
        input_output_aliases={4: 1},
    )(nfeat, agg2, flow, efeat, efeat_sc, w1h, w1l, w1h, w1l, w1h, w1l,
      b1, g1, be1, w2h, w2l, b2, g2, be2, w3h, w3l, b3)


def _split_bf16(w):
    wh = w.astype(jnp.bfloat16)
    wl = (w - wh.astype(jnp.float32)).astype(jnp.bfloat16)
    return wh, wl


def kernel(efeat, nfeat, flow_features, edge_index,
           W1, b1, g1, be1, W2, b2, g2, be2, W3, b3):
    zeros = jnp.zeros((ZROWS, D_E), jnp.float32)
    agg2, efeat_sc = _segment_sum_sc(efeat, edge_index.astype(jnp.int32), zeros)
    w1h, w1l = _split_bf16(W1)
    w2h, w2l = _split_bf16(W2)
    w3h, w3l = _split_bf16(W3)
    r = lambda v: v.reshape(1, -1)
    nfeat_new, efeat_out = _mlp_tc(
        nfeat, agg2, flow_features, efeat, efeat_sc,
        w1h, w1l, r(b1), r(g1), r(be1),
        w2h, w2l, r(b2), r(g2), r(be2), w3h, w3l, r(b3),
    )
    return (efeat_out, nfeat_new)


# speedup vs baseline: 1.1774x; 1.1774x over previous
import functools

import jax
import jax.numpy as jnp
from jax import lax
from jax.experimental import pallas as pl
from jax.experimental.pallas import tpu as pltpu
from jax.experimental.pallas import tpu_sc as plsc

N = 10000
E = 320000
D_N = 128
D_E = 128
D_F = 16
H = 256
D_OUT = 128

NC = 2
NS = 16
CHUNK = 128
ROWS_PER_TILE = 624
TAIL_ROWS = N - NS * ROWS_PER_TILE

NCHUNKS = E // CHUNK
CHUNK_BASE = NCHUNKS // (NC * NS)
CHUNK_EXTRA = NCHUNKS - CHUNK_BASE * NC * NS
ZROWS = 640


def _segment_sum_sc(efeat, edge_index, zeros):
    mesh = plsc.VectorSubcoreMesh(
        core_axis_name="c", subcore_axis_name="s", num_cores=NC, num_subcores=NS
    )

    @functools.partial(
        pl.kernel,
        out_type=(
            jax.ShapeDtypeStruct((NC, N, D_E), jnp.float32),
            jax.ShapeDtypeStruct((E, D_E), jnp.float32),
        ),
        mesh=mesh,
        scratch_types=[
            pltpu.VMEM_SHARED((N, D_E), jnp.float32),
            pltpu.VMEM((CHUNK, D_E), jnp.float32),
            pltpu.VMEM((CHUNK, D_E), jnp.float32),
            pltpu.VMEM((2, CHUNK), jnp.int32),
            pltpu.VMEM((2, CHUNK), jnp.int32),
            pltpu.SemaphoreType.DMA,
            pltpu.SemaphoreType.DMA,
            pltpu.SemaphoreType.DMA,
            pltpu.SemaphoreType.DMA,
            pltpu.SemaphoreType.DMA,
            pltpu.SemaphoreType.DMA,
            pltpu.SemaphoreType.DMA,
            pltpu.SemaphoreType.DMA,
        ],
    )
    def k(efeat_hbm, ei_hbm, z_hbm, out_hbm, eout_hbm, acc,
          rows0, rows1, idx0, idx1, sr0, sr1, si0, si1, sw0, sw1, ss0, ss1):
        cid = lax.axis_index("c")
        sid = lax.axis_index("s")
        w = cid * NS + sid
        r0 = sid * ROWS_PER_TILE
        pltpu.sync_copy(
            z_hbm.at[pl.ds(0, ROWS_PER_TILE)], acc.at[pl.ds(r0, ROWS_PER_TILE)]
        )

        @pl.when(sid == NS - 1)
        def _():
            pltpu.sync_copy(
                z_hbm.at[pl.ds(0, TAIL_ROWS)],
                acc.at[pl.ds(NS * ROWS_PER_TILE, TAIL_ROWS)],
            )

        nc_w = CHUNK_BASE + jnp.where(w < CHUNK_EXTRA, 1, 0)
        eb = (CHUNK_BASE * w + jnp.minimum(w, CHUNK_EXTRA)) * CHUNK

        def start(i, rows, idx, sr, si):
            b = eb + i * CHUNK
            pltpu.async_copy(efeat_hbm.at[pl.ds(b, CHUNK)], rows, sr)
            pltpu.async_copy(ei_hbm.at[pl.ds(0, 2), pl.ds(b, CHUNK)], idx, si)

        def wait(rows, idx, sr, si):
            pltpu.make_async_copy(efeat_hbm.at[pl.ds(0, CHUNK)], rows, sr).wait()
            pltpu.make_async_copy(
                ei_hbm.at[pl.ds(0, 2), pl.ds(0, CHUNK)], idx, si
            ).wait()

        def sc_start(rows, idx, ss):
            pltpu.async_copy(rows, acc.at[idx.at[1]], ss, add=True)

        def sc_wait(rows, idx, ss):
            pltpu.make_async_copy(rows, acc.at[idx.at[1]], ss).wait()

        def wb_start(i, rows, sw):
            pltpu.async_copy(rows, eout_hbm.at[pl.ds(eb + i * CHUNK, CHUNK)], sw)

        def wb_wait(rows, sw):
            pltpu.make_async_copy(rows, eout_hbm.at[pl.ds(0, CHUNK)], sw).wait()

        start(0, rows0, idx0, sr0, si0)
        plsc.subcore_barrier()

        @pl.loop(0, nc_w // 2)
        def _(j):
            i0 = 2 * j

            @pl.when(j > 0)
            def _():
                sc_wait(rows1, idx1, ss1)
                wb_wait(rows1, sw1)

            start(i0 + 1, rows1, idx1, sr1, si1)
            wait(rows0, idx0, sr0, si0)
            sc_start(rows0, idx0, ss0)
            wb_start(i0, rows0, sw0)
            wait(rows1, idx1, sr1, si1)
            sc_start(rows1, idx1, ss1)
            wb_start(i0 + 1, rows1, sw1)
            sc_wait(rows0, idx0, ss0)
            wb_wait(rows0, sw0)

            @pl.when(i0 + 2 < nc_w)
            def _():
                start(i0 + 2, rows0, idx0, sr0, si0)

        sc_wait(rows1, idx1, ss1)
        wb_wait(rows1, sw1)

        @pl.when(nc_w % 2 == 1)
        def _():
            wait(rows0, idx0, sr0, si0)
            sc_start(rows0, idx0, ss0)
            wb_start(nc_w - 1, rows0, sw0)
            sc_wait(rows0, idx0, ss0)
            wb_wait(rows0, sw0)

        plsc.subcore_barrier()
        pltpu.sync_copy(
            acc.at[pl.ds(r0, ROWS_PER_TILE)],
            out_hbm.at[cid, pl.ds(r0, ROWS_PER_TILE)],
        )

        @pl.when(sid == NS - 1)
        def _():
            pltpu.sync_copy(
                acc.at[pl.ds(NS * ROWS_PER_TILE, TAIL_ROWS)],
                out_hbm.at[cid, pl.ds(NS * ROWS_PER_TILE, TAIL_ROWS)],
            )

    return k(efeat, edge_index, zeros)


BR = 2000


def _ln(x, g, b, eps=1e-5):
    mu = jnp.mean(x, axis=-1, keepdims=True)
    var = jnp.mean((x - mu) ** 2, axis=-1, keepdims=True)
    return (x - mu) / jnp.sqrt(var + eps) * g + b


def _silu(x):
    return x / (1.0 + jnp.exp(-x))


def _dot3(x, wh, wl):
    xh = x.astype(jnp.bfloat16)
    xl = (x - xh.astype(jnp.float32)).astype(jnp.bfloat16)
    d = lambda a, b: jax.lax.dot_general(
        a, b, (((1,), (0,)), ((), ())), preferred_element_type=jnp.float32
    )
    return d(xh, wh) + d(xh, wl) + d(xl, wh)


def _mlp_body(nf, agg2, fl, w1nh, w1nl, w1eh, w1el, w1fh, w1fl, b1, g1, be1,
              w2h, w2l, b2, g2, be2, w3h, w3l, b3, out):
    x_n = nf[...]
    agg = agg2[0] + agg2[1]
    h = (
        _dot3(x_n, w1nh[...], w1nl[...])
        + _dot3(agg, w1eh[...], w1el[...])
        + _dot3(fl[...], w1fh[...], w1fl[...])
        + b1[...]
    )
    h = _silu(_ln(h, g1[...], be1[...]))
    h = _dot3(h, w2h[...], w2l[...]) + b2[...]
    h = _silu(_ln(h, g2[...], be2[...]))
    out[...] = _dot3(h, w3h[...], w3l[...]) + b3[...] + x_n


def _mlp_tc(nfeat, agg2, flow, w1h, w1l, b1, g1, be1, w2h, w2l, b2, g2, be2,
            w3h, w3l, b3):
    row_block = lambda d: pl.BlockSpec((BR, d), lambda i: (i, 0))
    full = lambda s: pl.BlockSpec(s, lambda i: (0, 0))
    w1_slabs = [
        pl.BlockSpec((D_N, H), lambda i: (0, 0)),
        pl.BlockSpec((D_E, H), lambda i: (1, 0)),
        pl.BlockSpec((D_F, H), lambda i: ((D_N + D_E) // D_F, 0)),
    ]
    return pl.pallas_call(
        _mlp_body,
        grid=(N // BR,),
        in_specs=[
            row_block(D_N),
            pl.BlockSpec((NC, BR, D_E), lambda i: (0, i, 0)),
            row_block(D_F),
            w1_slabs[0], w1_slabs[0],
            w1_slabs[1], w1_slabs[1],
            w1_slabs[2], w1_slabs[2],
            full((1, H)),
            full((1, H)),
            full((1, H)),
            full((H, H)), full((H, H)),
            full((1, H)),
            full((1, H)),
            full((1, H)),
            full((H, D_OUT)), full((H, D_OUT)),
            full((1, D_OUT)),
        ],
        out_specs=row_block(D_OUT),
        out_shape=jax.ShapeDtypeStruct((N, D_OUT), jnp.float32),
    )(nfeat, agg2, flow, w1h, w1l, w1h, w1l, w1h, w1l, b1, g1, be1,
      w2h, w2l, b2, g2, be2, w3h, w3l, b3)


def _split_bf16(w):
    wh = w.astype(jnp.bfloat16)
    wl = (w - wh.astype(jnp.float32)).astype(jnp.bfloat16)
    return wh, wl


def kernel(efeat, nfeat, flow_features, edge_index,
           W1, b1, g1, be1, W2, b2, g2, be2, W3, b3):
    zeros = jnp.zeros((ZROWS, D_E), jnp.float32)
    agg2, efeat_out = _segment_sum_sc(efeat, edge_index.astype(jnp.int32), zeros)
    w1h, w1l = _split_bf16(W1)
    w2h, w2l = _split_bf16(W2)
    w3h, w3l = _split_bf16(W3)
    r = lambda v: v.reshape(1, -1)
    nfeat_new = _mlp_tc(
        nfeat, agg2, flow_features,
        w1h, w1l, r(b1), r(g1), r(be1),
        w2h, w2l, r(b2), r(g2), r(be2), w3h, w3l, r(b3),
    )
    return (efeat_out, nfeat_new)
